# baseline (speedup 1.0000x reference)
import functools

import jax
import jax.numpy as jnp
from jax import lax
from jax.experimental import pallas as pl
from jax.experimental.pallas import tpu as pltpu
from jax.experimental.pallas import tpu_sc as plsc


def _im2col(x, ho, wo):
    b = x.shape[0]
    c = x.shape[3]
    cols = []
    for dy in range(3):
        for dx in range(3):
            cols.append(
                lax.slice(
                    x,
                    (0, dy, dx, 0),
                    (b, dy + 2 * ho - 1, dx + 2 * wo - 1, c),
                    (1, 2, 2, 1),
                )
            )
    return jnp.concatenate(cols, axis=-1).reshape(b, ho * wo, 9 * c)


def _conv_body(x_ref, w_ref, b_ref, o_ref, *, relu):
    y = jnp.dot(x_ref[0].astype(jnp.bfloat16), w_ref[...].astype(jnp.bfloat16),
                preferred_element_type=jnp.float32)
    y = y + b_ref[...]
    if relu:
        y = jnp.maximum(y, 0.0)
    o_ref[0] = y


def _conv_mm(xp, w, b, tile_n, relu):
    bsz, n, k = xp.shape
    cout = w.shape[0]
    wt = jnp.transpose(w, (2, 3, 1, 0)).reshape(k, cout)
    return pl.pallas_call(
        functools.partial(_conv_body, relu=relu),
        grid=(bsz, n // tile_n),
        in_specs=[
            pl.BlockSpec((1, tile_n, k), lambda bi, j: (bi, j, 0)),
            pl.BlockSpec((k, cout), lambda bi, j: (0, 0)),
            pl.BlockSpec((1, cout), lambda bi, j: (0, 0)),
        ],
        out_specs=pl.BlockSpec((1, tile_n, cout), lambda bi, j: (bi, j, 0)),
        out_shape=jax.ShapeDtypeStruct((bsz, n, cout), jnp.float32),
    )(xp, wt, b.reshape(1, cout))


def _vq_body(f_ref, cbt_ref, tok_ref):
    f = f_ref[0]
    cbt = cbt_ref[...]
    scores = jnp.dot(f.astype(jnp.bfloat16), cbt.astype(jnp.bfloat16),
                     preferred_element_type=jnp.float32)
    cbsq = jnp.sum(cbt * cbt, axis=0, keepdims=True)
    fsq = jnp.sum(f * f, axis=1, keepdims=True)
    sq = jnp.maximum(fsq + cbsq - 2.0 * scores, 0.0)
    m = jnp.min(sq, axis=1, keepdims=True)
    ki = lax.broadcasted_iota(jnp.int32, sq.shape, 1)
    idx = jnp.min(jnp.where(sq <= m, ki, jnp.int32(2 ** 30)), axis=1,
                  keepdims=True)
    tok_ref[0] = idx


def _vq_tokens(feats, codebook, tile_n):
    bsz, n, c = feats.shape
    k = codebook.shape[0]
    cbt = codebook.T
    toks = pl.pallas_call(
        _vq_body,
        grid=(bsz, n // tile_n),
        in_specs=[
            pl.BlockSpec((1, tile_n, c), lambda bi, j: (bi, j, 0)),
            pl.BlockSpec((c, k), lambda bi, j: (0, 0)),
        ],
        out_specs=pl.BlockSpec((1, tile_n, 1), lambda bi, j: (bi, j, 0)),
        out_shape=jax.ShapeDtypeStruct((bsz, n, 1), jnp.int32),
    )(feats, cbt)
    return toks.reshape(bsz, n)


_NC = 2
_NS = 16
_NW = _NC * _NS
_CHUNK = 96
_NCHUNK = 3


def _sc_gather_body(emb_hbm, idx_hbm, out_hbm, idx_v, rows_v, sem):
    wid = lax.axis_index("s") * _NC + lax.axis_index("c")
    base = wid * (_CHUNK * _NCHUNK)
    for ch in range(_NCHUNK):
        off = base + ch * _CHUNK
        pltpu.sync_copy(idx_hbm.at[pl.ds(off, _CHUNK)], idx_v.at[ch])
        pltpu.async_copy(emb_hbm.at[idx_v.at[ch]], rows_v, sem).wait()
        pltpu.sync_copy(rows_v, out_hbm.at[pl.ds(off, _CHUNK)])


def _sc_gather(embedding, tokens_flat):
    total = tokens_flat.shape[0]
    d = embedding.shape[1]
    assert d % 128 == 0 and total == _NW * _CHUNK * _NCHUNK
    mesh = plsc.VectorSubcoreMesh(core_axis_name="c", subcore_axis_name="s")
    fn = functools.partial(
        pl.kernel,
        mesh=mesh,
        out_type=jax.ShapeDtypeStruct((total, d), jnp.float32),
        scratch_types=[
            pltpu.VMEM((_NCHUNK, _CHUNK), jnp.int32),
            pltpu.VMEM((_CHUNK, d), jnp.float32),
            pltpu.SemaphoreType.DMA,
        ],
    )(_sc_gather_body)
    return fn(embedding, tokens_flat)


def kernel(image, w1, b1, w2, b2, w3, b3, codebook, embedding,
           resolution_level=0):
    bsz = image.shape[0]
    h = image.shape[2]
    h1, h2, h3 = h // 2, h // 4, h // 8

    x = jnp.transpose(image, (0, 2, 3, 1))
    x = jnp.pad(x, ((0, 0), (1, 1), (1, 1), (0, 0)))
    a1 = _conv_mm(_im2col(x, h1, h1), w1, b1, tile_n=4608, relu=True)

    x = a1.reshape(bsz, h1, h1, w1.shape[0])
    x = jnp.pad(x, ((0, 0), (1, 1), (1, 1), (0, 0)))
    a2 = _conv_mm(_im2col(x, h2, h2), w2, b2, tile_n=2304, relu=True)

    x = a2.reshape(bsz, h2, h2, w2.shape[0])
    x = jnp.pad(x, ((0, 0), (1, 1), (1, 1), (0, 0)))
    feats = _conv_mm(_im2col(x, h3, h3), w3, b3, tile_n=2304, relu=False)

    tokens = _vq_tokens(feats, codebook, tile_n=256)

    d = embedding.shape[1]
    emb_pad = jnp.pad(embedding, ((0, 0), (0, (-d) % 128)))
    emb = _sc_gather(emb_pad, tokens.reshape(-1))
    emb = emb[:, :d].reshape(bsz, h3 * h3, d)
    return tokens, emb


# speedup vs baseline: 1.3696x; 1.3696x over previous
import functools

import jax
import jax.numpy as jnp
from jax import lax
from jax.experimental import pallas as pl
from jax.experimental.pallas import tpu as pltpu
from jax.experimental.pallas import tpu_sc as plsc


def _im2col(x, ho, wo):
    b, c = x.shape[0], x.shape[1]
    cols = []
    for dy in range(3):
        for dx in range(3):
            cols.append(
                lax.slice(
                    x,
                    (0, 0, dy, dx),
                    (b, c, dy + 2 * ho - 1, dx + 2 * wo - 1),
                    (1, 1, 2, 2),
                )
            )
    return jnp.concatenate(cols, axis=1).reshape(b, 9 * c, ho * wo)


def _conv_body(x_ref, w_ref, b_ref, o_ref, *, relu):
    y = jnp.dot(w_ref[...].astype(jnp.bfloat16), x_ref[0].astype(jnp.bfloat16),
                preferred_element_type=jnp.float32)
    y = y + b_ref[...]
    if relu:
        y = jnp.maximum(y, 0.0)
    o_ref[0] = y


def _conv_mm(xp, w, b, tile_n, relu):
    bsz, k, n = xp.shape
    cout = w.shape[0]
    wt = jnp.transpose(w, (0, 2, 3, 1)).reshape(cout, k)
    return pl.pallas_call(
        functools.partial(_conv_body, relu=relu),
        grid=(bsz, n // tile_n),
        in_specs=[
            pl.BlockSpec((1, k, tile_n), lambda bi, j: (bi, 0, j)),
            pl.BlockSpec((cout, k), lambda bi, j: (0, 0)),
            pl.BlockSpec((cout, 1), lambda bi, j: (0, 0)),
        ],
        out_specs=pl.BlockSpec((1, cout, tile_n), lambda bi, j: (bi, 0, j)),
        out_shape=jax.ShapeDtypeStruct((bsz, cout, n), jnp.float32),
    )(xp, wt, b.reshape(cout, 1))


def _vq_body(f_ref, cbt_ref, tok_ref):
    f = f_ref[0]
    cbt = cbt_ref[...]
    scores = jnp.dot(f.astype(jnp.bfloat16), cbt.astype(jnp.bfloat16),
                     preferred_element_type=jnp.float32)
    cbsq = jnp.sum(cbt * cbt, axis=0, keepdims=True)
    fsq = jnp.sum(f * f, axis=1, keepdims=True)
    sq = jnp.maximum(fsq + cbsq - 2.0 * scores, 0.0)
    m = jnp.min(sq, axis=1, keepdims=True)
    ki = lax.broadcasted_iota(jnp.int32, sq.shape, 1)
    idx = jnp.min(jnp.where(sq <= m, ki, jnp.int32(2 ** 30)), axis=1,
                  keepdims=True)
    tok_ref[0] = idx


def _vq_tokens(feats, codebook, tile_n):
    bsz, n, c = feats.shape
    k = codebook.shape[0]
    cbt = codebook.T
    toks = pl.pallas_call(
        _vq_body,
        grid=(bsz, n // tile_n),
        in_specs=[
            pl.BlockSpec((1, tile_n, c), lambda bi, j: (bi, j, 0)),
            pl.BlockSpec((c, k), lambda bi, j: (0, 0)),
        ],
        out_specs=pl.BlockSpec((1, tile_n, 1), lambda bi, j: (bi, j, 0)),
        out_shape=jax.ShapeDtypeStruct((bsz, n, 1), jnp.int32),
    )(feats, cbt)
    return toks.reshape(bsz, n)


_NC = 2
_NS = 16
_NW = _NC * _NS
_CHUNK = 96
_NCHUNK = 3


def _sc_gather_body(emb_hbm, idx_hbm, out_hbm, idx_v, rows_v, sem):
    wid = lax.axis_index("s") * _NC + lax.axis_index("c")
    base = wid * (_CHUNK * _NCHUNK)
    for ch in range(_NCHUNK):
        off = base + ch * _CHUNK
        pltpu.sync_copy(idx_hbm.at[pl.ds(off, _CHUNK)], idx_v.at[ch])
        pltpu.async_copy(emb_hbm.at[idx_v.at[ch]], rows_v, sem).wait()
        pltpu.sync_copy(rows_v, out_hbm.at[pl.ds(off, _CHUNK)])


def _sc_gather(embedding, tokens_flat):
    total = tokens_flat.shape[0]
    d = embedding.shape[1]
    assert d % 128 == 0 and total == _NW * _CHUNK * _NCHUNK
    mesh = plsc.VectorSubcoreMesh(core_axis_name="c", subcore_axis_name="s")
    fn = functools.partial(
        pl.kernel,
        mesh=mesh,
        out_type=jax.ShapeDtypeStruct((total, d), jnp.float32),
        scratch_types=[
            pltpu.VMEM((_NCHUNK, _CHUNK), jnp.int32),
            pltpu.VMEM((_CHUNK, d), jnp.float32),
            pltpu.SemaphoreType.DMA,
        ],
    )(_sc_gather_body)
    return fn(embedding, tokens_flat)


def kernel(image, w1, b1, w2, b2, w3, b3, codebook, embedding,
           resolution_level=0):
    bsz = image.shape[0]
    h = image.shape[2]
    h1, h2, h3 = h // 2, h // 4, h // 8

    x = jnp.pad(image, ((0, 0), (0, 0), (1, 1), (1, 1)))
    a1 = _conv_mm(_im2col(x, h1, h1), w1, b1, tile_n=4608, relu=True)

    x = a1.reshape(bsz, w1.shape[0], h1, h1)
    x = jnp.pad(x, ((0, 0), (0, 0), (1, 1), (1, 1)))
    a2 = _conv_mm(_im2col(x, h2, h2), w2, b2, tile_n=2304, relu=True)

    x = a2.reshape(bsz, w2.shape[0], h2, h2)
    x = jnp.pad(x, ((0, 0), (0, 0), (1, 1), (1, 1)))
    a3 = _conv_mm(_im2col(x, h3, h3), w3, b3, tile_n=2304, relu=False)

    feats = jnp.transpose(a3, (0, 2, 1))
    tokens = _vq_tokens(feats, codebook, tile_n=256)

    d = embedding.shape[1]
    emb_pad = jnp.pad(embedding, ((0, 0), (0, (-d) % 128)))
    emb = _sc_gather(emb_pad, tokens.reshape(-1))
    emb = emb[:, :d].reshape(bsz, h3 * h3, d)
    return tokens, emb


# speedup vs baseline: 6.8663x; 5.0132x over previous
import functools

import jax
import jax.numpy as jnp
from jax import lax
from jax.experimental import pallas as pl
from jax.experimental.pallas import tpu as pltpu
from jax.experimental.pallas import tpu_sc as plsc


def _conv_layer_body(x_ref, w_ref, b_ref, o_ref, *, c, ho, wo, wp, wop,
                     cout, pad_out, relu):
    x = x_ref[0]
    hp = x.shape[1]
    xr = x.reshape(c, hp // 2, 2, wp)
    pe = xr[:, :, 0, :]
    po = xr[:, :, 1, :]
    rows = (pe[:, 0:ho], po[:, 0:ho], pe[:, 1:ho + 1])

    ii = lax.broadcasted_iota(jnp.int32, (wp, 3 * wop), 0)
    jj = lax.broadcasted_iota(jnp.int32, (wp, 3 * wop), 1)
    dx = jj // wop
    xq = jj % wop
    sel = ((ii == 2 * xq + dx) & (xq < wo)).astype(jnp.bfloat16)

    taps = []
    for dy in range(3):
        r = rows[dy].reshape(c * ho, wp)
        m = jnp.dot(r.astype(jnp.bfloat16), sel,
                    preferred_element_type=jnp.float32)
        for dxi in range(3):
            taps.append(m[:, dxi * wop:(dxi + 1) * wop].reshape(c, ho, wop))
    p = jnp.concatenate(taps, axis=0).reshape(9 * c, ho * wop)

    y = jnp.dot(w_ref[...].astype(jnp.bfloat16), p.astype(jnp.bfloat16),
                preferred_element_type=jnp.float32)
    y = y + b_ref[...]
    if relu:
        y = jnp.maximum(y, 0.0)
    y = y.reshape(cout, ho, wop)
    if pad_out:
        o_ref[0] = jnp.zeros(o_ref.shape[1:], jnp.float32)
        o_ref[0, :, 1:ho + 1, 1:wo + 1] = y[:, :, 0:wo]
    else:
        o_ref[0] = y


def _conv_layer(xp, w, b, ho, wo, wop, wnext, relu, pad_out):
    bsz, c, hp, wp = xp.shape
    cout = w.shape[0]
    wt = jnp.transpose(w, (2, 3, 1, 0)).reshape(9 * c, cout).T
    if pad_out:
        oshape = (bsz, cout, ho + 2, wnext)
    else:
        oshape = (bsz, cout, ho, wop)
    return pl.pallas_call(
        functools.partial(_conv_layer_body, c=c, ho=ho, wo=wo, wp=wp,
                          wop=wop, cout=cout, pad_out=pad_out, relu=relu),
        grid=(bsz,),
        in_specs=[
            pl.BlockSpec((1, c, hp, wp), lambda bi: (bi, 0, 0, 0)),
            pl.BlockSpec((cout, 9 * c), lambda bi: (0, 0)),
            pl.BlockSpec((cout, 1), lambda bi: (0, 0)),
        ],
        out_specs=pl.BlockSpec((1,) + oshape[1:], lambda bi: (bi, 0, 0, 0)),
        out_shape=jax.ShapeDtypeStruct(oshape, jnp.float32),
    )(xp, wt, b.reshape(cout, 1))


def _vq_body(f_ref, cb_ref, tok_ref):
    f = f_ref[0]
    cb = cb_ref[...]
    scores = jnp.dot(cb.astype(jnp.bfloat16), f.astype(jnp.bfloat16),
                     preferred_element_type=jnp.float32)
    cbsq = jnp.sum(cb * cb, axis=1, keepdims=True)
    fsq = jnp.sum(f * f, axis=0, keepdims=True)
    sq = jnp.maximum(cbsq + fsq - 2.0 * scores, 0.0)
    m = jnp.min(sq, axis=0, keepdims=True)
    ki = lax.broadcasted_iota(jnp.int32, sq.shape, 0)
    idx = jnp.min(jnp.where(sq <= m, ki, jnp.int32(2 ** 30)), axis=0,
                  keepdims=True)
    tok_ref[0] = idx


def _vq_tokens(feats_cm, codebook, tile_n):
    bsz, c, n = feats_cm.shape
    k = codebook.shape[0]
    toks = pl.pallas_call(
        _vq_body,
        grid=(bsz, n // tile_n),
        in_specs=[
            pl.BlockSpec((1, c, tile_n), lambda bi, j: (bi, 0, j)),
            pl.BlockSpec((k, c), lambda bi, j: (0, 0)),
        ],
        out_specs=pl.BlockSpec((1, 1, tile_n), lambda bi, j: (bi, 0, j)),
        out_shape=jax.ShapeDtypeStruct((bsz, 1, n), jnp.int32),
    )(feats_cm, codebook)
    return toks.reshape(bsz, n)


_NC = 2
_NS = 16
_NW = _NC * _NS
_CHUNK = 96
_NCHUNK = 3
_BPW = _CHUNK * _NCHUNK


def _sc_gather_body(emb_hbm, idx_hbm, out_hbm, idx_v, rows_v, sem):
    wid = lax.axis_index("s") * _NC + lax.axis_index("c")
    base = wid * _BPW
    for ch in range(_NCHUNK):
        pltpu.sync_copy(idx_hbm.at[pl.ds(base + ch * _CHUNK, _CHUNK)],
                        idx_v.at[ch])
    copies = []
    for ch in range(_NCHUNK):
        copies.append(pltpu.async_copy(
            emb_hbm.at[idx_v.at[ch]],
            rows_v.at[pl.ds(ch * _CHUNK, _CHUNK)], sem))
    for cp in copies:
        cp.wait()
    pltpu.sync_copy(rows_v, out_hbm.at[pl.ds(base, _BPW)])


def _sc_gather(embedding, tokens_flat):
    total = tokens_flat.shape[0]
    d = embedding.shape[1]
    assert d % 128 == 0 and total == _NW * _BPW
    mesh = plsc.VectorSubcoreMesh(core_axis_name="c", subcore_axis_name="s")
    fn = functools.partial(
        pl.kernel,
        mesh=mesh,
        out_type=jax.ShapeDtypeStruct((total, d), jnp.float32),
        scratch_types=[
            pltpu.VMEM((_NCHUNK, _CHUNK), jnp.int32),
            pltpu.VMEM((_BPW, d), jnp.float32),
            pltpu.SemaphoreType.DMA,
        ],
    )(_sc_gather_body)
    return fn(embedding, tokens_flat)


def kernel(image, w1, b1, w2, b2, w3, b3, codebook, embedding,
           resolution_level=0):
    bsz = image.shape[0]
    h = image.shape[2]
    h1, h2, h3 = h // 2, h // 4, h // 8

    xp = jnp.pad(image, ((0, 0), (0, 0), (1, 1), (1, 127)))
    a1 = _conv_layer(xp, w1, b1, ho=h1, wo=h1, wop=256, wnext=256,
                     relu=True, pad_out=True)
    a2 = _conv_layer(a1, w2, b2, ho=h2, wo=h2, wop=128, wnext=128,
                     relu=True, pad_out=True)
    a3 = _conv_layer(a2, w3, b3, ho=h3, wo=h3, wop=128, wnext=0,
                     relu=False, pad_out=False)

    feats_cm = a3[:, :, :, :h3].reshape(bsz, 64, h3 * h3)
    tokens = _vq_tokens(feats_cm, codebook, tile_n=256)

    d = embedding.shape[1]
    emb_pad = jnp.pad(embedding, ((0, 0), (0, (-d) % 128)))
    emb = _sc_gather(emb_pad, tokens.reshape(-1))
    emb = emb[:, :d].reshape(bsz, h3 * h3, d)
    return tokens, emb


# speedup vs baseline: 6.9376x; 1.0104x over previous
import functools

import jax
import jax.numpy as jnp
from jax import lax
from jax.experimental import pallas as pl
from jax.experimental.pallas import tpu as pltpu
from jax.experimental.pallas import tpu_sc as plsc


def _conv_layer_body(x_ref, w_ref, b_ref, o_ref, *, c, ho, wo, wp, wop,
                     cout, pad_out, relu):
    x = x_ref[0]
    hp = x.shape[1]
    xr = x.reshape(c, hp // 2, 2, wp)
    pe = xr[:, :, 0, :]
    po = xr[:, :, 1, :]
    rows = (pe[:, 0:ho], po[:, 0:ho], pe[:, 1:ho + 1])

    ii = lax.broadcasted_iota(jnp.int32, (wp, 3 * wop), 0)
    jj = lax.broadcasted_iota(jnp.int32, (wp, 3 * wop), 1)
    dx = jj // wop
    xq = jj % wop
    sel = ((ii == 2 * xq + dx) & (xq < wo)).astype(jnp.bfloat16)

    taps = []
    for dy in range(3):
        r = rows[dy].reshape(c * ho, wp)
        m = jnp.dot(r.astype(jnp.bfloat16), sel,
                    preferred_element_type=jnp.float32)
        for dxi in range(3):
            taps.append(m[:, dxi * wop:(dxi + 1) * wop].reshape(c, ho, wop))
    p = jnp.concatenate(taps, axis=0).reshape(9 * c, ho * wop)

    y = jnp.dot(w_ref[...].astype(jnp.bfloat16), p.astype(jnp.bfloat16),
                preferred_element_type=jnp.float32)
    y = y + b_ref[...]
    if relu:
        y = jnp.maximum(y, 0.0)
    y = y.reshape(cout, ho, wop)
    if pad_out:
        o_ref[0] = jnp.zeros(o_ref.shape[1:], jnp.float32)
        o_ref[0, :, 1:ho + 1, 1:wo + 1] = y[:, :, 0:wo]
    else:
        o_ref[0] = y


def _conv_layer(xp, w, b, ho, wo, wop, wnext, relu, pad_out):
    bsz, c, hp, wp = xp.shape
    cout = w.shape[0]
    wt = jnp.transpose(w, (2, 3, 1, 0)).reshape(9 * c, cout).T
    if pad_out:
        oshape = (bsz, cout, ho + 2, wnext)
    else:
        oshape = (bsz, cout, ho, wop)
    return pl.pallas_call(
        functools.partial(_conv_layer_body, c=c, ho=ho, wo=wo, wp=wp,
                          wop=wop, cout=cout, pad_out=pad_out, relu=relu),
        grid=(bsz,),
        in_specs=[
            pl.BlockSpec((1, c, hp, wp), lambda bi: (bi, 0, 0, 0)),
            pl.BlockSpec((cout, 9 * c), lambda bi: (0, 0)),
            pl.BlockSpec((cout, 1), lambda bi: (0, 0)),
        ],
        out_specs=pl.BlockSpec((1,) + oshape[1:], lambda bi: (bi, 0, 0, 0)),
        out_shape=jax.ShapeDtypeStruct(oshape, jnp.float32),
    )(xp, wt, b.reshape(cout, 1))


def _vq_body(f_ref, cb_ref, tok_ref):
    f = f_ref[0]
    cb = cb_ref[...]
    scores = jnp.dot(cb.astype(jnp.bfloat16), f.astype(jnp.bfloat16),
                     preferred_element_type=jnp.float32)
    cbsq = jnp.sum(cb * cb, axis=1, keepdims=True)
    fsq = jnp.sum(f * f, axis=0, keepdims=True)
    sq = jnp.maximum(cbsq + fsq - 2.0 * scores, 0.0)
    m = jnp.min(sq, axis=0, keepdims=True)
    ki = lax.broadcasted_iota(jnp.int32, sq.shape, 0)
    idx = jnp.min(jnp.where(sq <= m, ki, jnp.int32(2 ** 30)), axis=0,
                  keepdims=True)
    tok_ref[0] = idx


def _vq_tokens(feats_cm, codebook, tile_n):
    bsz, c, n = feats_cm.shape
    k = codebook.shape[0]
    toks = pl.pallas_call(
        _vq_body,
        grid=(bsz, n // tile_n),
        in_specs=[
            pl.BlockSpec((1, c, tile_n), lambda bi, j: (bi, 0, j)),
            pl.BlockSpec((k, c), lambda bi, j: (0, 0)),
        ],
        out_specs=pl.BlockSpec((1, 1, tile_n), lambda bi, j: (bi, 0, j)),
        out_shape=jax.ShapeDtypeStruct((bsz, 1, n), jnp.int32),
    )(feats_cm, codebook)
    return toks.reshape(bsz, n)


_NC = 2
_NS = 16
_NW = _NC * _NS
_CHUNK = 96
_NCHUNK = 3
_BPW = _CHUNK * _NCHUNK


def _sc_gather_body(emb_hbm, idx_hbm, out_hbm, idx_v, rows_v, sem):
    wid = lax.axis_index("s") * _NC + lax.axis_index("c")
    base = wid * _BPW
    for ch in range(_NCHUNK):
        pltpu.sync_copy(idx_hbm.at[pl.ds(base + ch * _CHUNK, _CHUNK)],
                        idx_v.at[ch])
    copies = []
    for ch in range(_NCHUNK):
        copies.append(pltpu.async_copy(
            emb_hbm.at[idx_v.at[ch]],
            rows_v.at[pl.ds(ch * _CHUNK, _CHUNK)], sem))
    for cp in copies:
        cp.wait()
    pltpu.sync_copy(rows_v, out_hbm.at[pl.ds(base, _BPW)])


def _sc_gather(embedding, tokens_flat):
    total = tokens_flat.shape[0]
    d = embedding.shape[1]
    assert d % 128 == 0 and total == _NW * _BPW
    mesh = plsc.VectorSubcoreMesh(core_axis_name="c", subcore_axis_name="s")
    fn = functools.partial(
        pl.kernel,
        mesh=mesh,
        out_type=jax.ShapeDtypeStruct((total, d), jnp.float32),
        scratch_types=[
            pltpu.VMEM((_NCHUNK, _CHUNK), jnp.int32),
            pltpu.VMEM((_BPW, d), jnp.float32),
            pltpu.SemaphoreType.DMA,
        ],
    )(_sc_gather_body)
    return fn(embedding, tokens_flat)


def kernel(image, w1, b1, w2, b2, w3, b3, codebook, embedding,
           resolution_level=0):
    bsz = image.shape[0]
    h = image.shape[2]
    h1, h2, h3 = h // 2, h // 4, h // 8

    xp = jnp.pad(image, ((0, 0), (0, 0), (1, 1), (1, 127)))
    a1 = _conv_layer(xp, w1, b1, ho=h1, wo=h1, wop=256, wnext=256,
                     relu=True, pad_out=True)
    a2 = _conv_layer(a1, w2, b2, ho=h2, wo=h2, wop=128, wnext=128,
                     relu=True, pad_out=True)
    a3 = _conv_layer(a2, w3, b3, ho=h3, wo=h3, wop=128, wnext=0,
                     relu=False, pad_out=False)

    feats_cm = a3[:, :, :, :h3].reshape(bsz, 64, h3 * h3)
    tokens = _vq_tokens(feats_cm, codebook, tile_n=384)

    d = embedding.shape[1]
    emb_pad = jnp.pad(embedding, ((0, 0), (0, (-d) % 128)))
    emb = _sc_gather(emb_pad, tokens.reshape(-1))
    emb = emb[:, :d].reshape(bsz, h3 * h3, d)
    return tokens, emb


# speedup vs baseline: 15.3791x; 2.2168x over previous
import functools

import jax
import jax.numpy as jnp
from jax import lax
from jax.experimental import pallas as pl
from jax.experimental.pallas import tpu as pltpu
from jax.experimental.pallas import tpu_sc as plsc


def _conv_layer_body(x_ref, w_ref, b_ref, o_ref, *, c, ho, wo, wp, wop,
                     cout, pad_out, relu):
    x = x_ref[0]
    hp = x.shape[1]
    xr = x.reshape(c, hp // 2, 2, wp)
    pe = xr[:, :, 0, :]
    po = xr[:, :, 1, :]
    rows = (pe[:, 0:ho], po[:, 0:ho], pe[:, 1:ho + 1])

    ii = lax.broadcasted_iota(jnp.int32, (wp, 3 * wop), 0)
    jj = lax.broadcasted_iota(jnp.int32, (wp, 3 * wop), 1)
    dx = jj // wop
    xq = jj % wop
    sel = ((ii == 2 * xq + dx) & (xq < wo)).astype(jnp.bfloat16)

    taps = []
    for dy in range(3):
        r = rows[dy].reshape(c * ho, wp)
        m = jnp.dot(r.astype(jnp.bfloat16), sel,
                    preferred_element_type=jnp.float32)
        for dxi in range(3):
            taps.append(m[:, dxi * wop:(dxi + 1) * wop].reshape(c, ho, wop))
    p = jnp.concatenate(taps, axis=0).reshape(9 * c, ho * wop)

    y = jnp.dot(w_ref[...].astype(jnp.bfloat16), p.astype(jnp.bfloat16),
                preferred_element_type=jnp.float32)
    y = y + b_ref[...]
    if relu:
        y = jnp.maximum(y, 0.0)
    y = y.reshape(cout, ho, wop)
    if pad_out:
        o_ref[0] = jnp.zeros(o_ref.shape[1:], jnp.float32)
        o_ref[0, :, 1:ho + 1, 1:wo + 1] = y[:, :, 0:wo]
    else:
        o_ref[0] = y


def _conv_layer(xp, w, b, ho, wo, wop, wnext, relu, pad_out):
    bsz, c, hp, wp = xp.shape
    cout = w.shape[0]
    wt = jnp.transpose(w, (2, 3, 1, 0)).reshape(9 * c, cout).T
    if pad_out:
        oshape = (bsz, cout, ho + 2, wnext)
    else:
        oshape = (bsz, cout, ho, wop)
    return pl.pallas_call(
        functools.partial(_conv_layer_body, c=c, ho=ho, wo=wo, wp=wp,
                          wop=wop, cout=cout, pad_out=pad_out, relu=relu),
        grid=(bsz,),
        in_specs=[
            pl.BlockSpec((1, c, hp, wp), lambda bi: (bi, 0, 0, 0)),
            pl.BlockSpec((cout, 9 * c), lambda bi: (0, 0)),
            pl.BlockSpec((cout, 1), lambda bi: (0, 0)),
        ],
        out_specs=pl.BlockSpec((1,) + oshape[1:], lambda bi: (bi, 0, 0, 0)),
        out_shape=jax.ShapeDtypeStruct(oshape, jnp.float32),
    )(xp, wt, b.reshape(cout, 1))


def _vq_body(f_ref, cb_ref, tok_ref):
    f = f_ref[0]
    cb = cb_ref[...]
    scores = jnp.dot(cb.astype(jnp.bfloat16), f.astype(jnp.bfloat16),
                     preferred_element_type=jnp.float32)
    cbsq = jnp.sum(cb * cb, axis=1, keepdims=True)
    fsq = jnp.sum(f * f, axis=0, keepdims=True)
    sq = jnp.maximum(cbsq + fsq - 2.0 * scores, 0.0)
    m = jnp.min(sq, axis=0, keepdims=True)
    ki = lax.broadcasted_iota(jnp.int32, sq.shape, 0)
    idx = jnp.min(jnp.where(sq <= m, ki, jnp.int32(2 ** 30)), axis=0,
                  keepdims=True)
    tok_ref[0] = idx


def _vq_tokens(feats_cm, codebook, tile_n):
    bsz, c, n = feats_cm.shape
    k = codebook.shape[0]
    toks = pl.pallas_call(
        _vq_body,
        grid=(bsz, n // tile_n),
        in_specs=[
            pl.BlockSpec((1, c, tile_n), lambda bi, j: (bi, 0, j)),
            pl.BlockSpec((k, c), lambda bi, j: (0, 0)),
        ],
        out_specs=pl.BlockSpec((1, 1, tile_n), lambda bi, j: (bi, 0, j)),
        out_shape=jax.ShapeDtypeStruct((bsz, 1, n), jnp.int32),
    )(feats_cm, codebook)
    return toks.reshape(bsz, n)


_NC = 2
_NS = 16
_NW = _NC * _NS
_CHUNK = 96
_NCHUNK = 3
_BPW = _CHUNK * _NCHUNK


def _sc_gather_body(emb_hbm, idx_hbm, out_hbm, idx_v, rows_v, tbl_sh, sem):
    sid = lax.axis_index("s")
    wid = sid * _NC + lax.axis_index("c")
    base = wid * _BPW

    @pl.when(sid == 0)
    def _stage():
        pltpu.sync_copy(emb_hbm, tbl_sh)
    plsc.subcore_barrier()

    for ch in range(_NCHUNK):
        pltpu.sync_copy(idx_hbm.at[pl.ds(base + ch * _CHUNK, _CHUNK)],
                        idx_v.at[ch])
    copies = []
    for ch in range(_NCHUNK):
        copies.append(pltpu.async_copy(
            tbl_sh.at[idx_v.at[ch]],
            rows_v.at[pl.ds(ch * _CHUNK, _CHUNK)], sem))
    for cp in copies:
        cp.wait()
    pltpu.sync_copy(rows_v, out_hbm.at[pl.ds(base, _BPW)])


def _sc_gather(embedding, tokens_flat):
    total = tokens_flat.shape[0]
    d = embedding.shape[1]
    assert d % 128 == 0 and total == _NW * _BPW
    mesh = plsc.VectorSubcoreMesh(core_axis_name="c", subcore_axis_name="s")
    fn = functools.partial(
        pl.kernel,
        mesh=mesh,
        out_type=jax.ShapeDtypeStruct((total, d), jnp.float32),
        scratch_types=[
            pltpu.VMEM((_NCHUNK, _CHUNK), jnp.int32),
            pltpu.VMEM((_BPW, d), jnp.float32),
            pltpu.VMEM_SHARED((8192, d), jnp.float32),
            pltpu.SemaphoreType.DMA,
        ],
    )(_sc_gather_body)
    return fn(embedding, tokens_flat)


def kernel(image, w1, b1, w2, b2, w3, b3, codebook, embedding,
           resolution_level=0):
    bsz = image.shape[0]
    h = image.shape[2]
    h1, h2, h3 = h // 2, h // 4, h // 8

    xp = jnp.pad(image, ((0, 0), (0, 0), (1, 1), (1, 127)))
    a1 = _conv_layer(xp, w1, b1, ho=h1, wo=h1, wop=256, wnext=256,
                     relu=True, pad_out=True)
    a2 = _conv_layer(a1, w2, b2, ho=h2, wo=h2, wop=128, wnext=128,
                     relu=True, pad_out=True)
    a3 = _conv_layer(a2, w3, b3, ho=h3, wo=h3, wop=128, wnext=0,
                     relu=False, pad_out=False)

    feats_cm = a3[:, :, :, :h3].reshape(bsz, 64, h3 * h3)
    tokens = _vq_tokens(feats_cm, codebook, tile_n=384)

    d = embedding.shape[1]
    emb_pad = jnp.pad(embedding, ((0, 0), (0, (-d) % 128)))
    emb = _sc_gather(emb_pad, tokens.reshape(-1))
    emb = emb[:, :d].reshape(bsz, h3 * h3, d)
    return tokens, emb
